# baseline (speedup 1.0000x reference)
import jax
import jax.numpy as jnp
from jax import lax
from jax.experimental import pallas as pl
from jax.experimental.pallas import tpu as pltpu
from jax.experimental.pallas import tpu_sc as plsc

B, L = 4096, 200
DIM = 64
NC, NS = 2, 16
NW = NC * NS
RPW = B // NW
LANES = 16
NCHUNK = L // LANES
NJ = DIM // LANES
WPR = L * DIM // 128
NPAIR = RPW // 2


def _zero_acc(in_v, b, acc):
    for c in range(NCHUNK):
        v = in_v[b, pl.ds(c * LANES, LANES)]
        acc = acc | jnp.where(v == 0, 1, 0)
    v = in_v[b, pl.ds(L - LANES, LANES)]
    acc = acc | jnp.where(v == 0, 1, 0)
    return acc


def _body(in_hbm, tab1_hbm, r0_hbm, out_hbm, in_v, d2_v, r0_v, scr_v, sem, sem_s):
    wid = lax.axis_index("s") * NC + lax.axis_index("c")
    base = wid * RPW

    pltpu.sync_copy(in_hbm.at[pl.ds(base, RPW)], in_v)
    pltpu.sync_copy(tab1_hbm, d2_v.at[pl.ds(0, WPR)])
    pltpu.sync_copy(tab1_hbm, d2_v.at[pl.ds(WPR, WPR)])
    pltpu.sync_copy(r0_hbm, r0_v)

    r0c = [r0_v[pl.ds(j * LANES, LANES)] for j in range(NJ)]

    def compose_row(b, half):
        rbase = half * WPR

        def blend_lane(vv, lane, srow):
            sc = jnp.where(vv[lane] == 0, 0.0, 1.0).astype(jnp.float32)
            zf = jnp.broadcast_to(sc, (LANES,))
            cb = (lane % 2) * DIM
            for j in range(NJ):
                col = cb + j * LANES
                dc = d2_v[srow, pl.ds(col, LANES)]
                scr_v[srow, pl.ds(col, LANES)] = r0c[j] + zf * (dc - r0c[j])

        def fix(c, carry):
            off = pl.multiple_of(c * LANES, LANES)
            v = in_v[b, pl.ds(off, LANES)]
            for lane in range(LANES):
                blend_lane(v, lane, rbase + (off + lane) // 2)
            return carry
        lax.fori_loop(0, NCHUNK, fix, 0)
        vt = in_v[b, pl.ds(L - LANES, LANES)]
        for lane in range(L - NCHUNK * LANES, LANES):
            blend_lane(vt, lane, rbase + (L - LANES + lane) // 2)

    def pair(p, cnt):
        b0 = 2 * p
        acc = jnp.zeros((LANES,), jnp.int32)
        acc = _zero_acc(in_v, b0, acc)
        acc = _zero_acc(in_v, b0 + 1, acc)
        s = acc[0]
        for i in range(1, LANES):
            s = s | acc[i]

        dst = out_hbm.at[pl.ds((base + b0) * WPR, 2 * WPR)]

        def fast(c):
            pltpu.make_async_copy(d2_v, dst, sem).start()
            return c + 1

        def slow(c):
            compose_row(b0, 0)
            compose_row(b0 + 1, 1)
            cp = pltpu.make_async_copy(scr_v, dst, sem_s)
            cp.start()
            cp.wait()
            return c

        return lax.cond(s == 0, fast, slow, cnt)

    cnt = lax.fori_loop(0, NPAIR, pair, jnp.int32(0))

    def drain(i, carry):
        pltpu.make_async_copy(d2_v, out_hbm.at[pl.ds(0, 2 * WPR)], sem).wait()
        return carry

    lax.fori_loop(0, cnt, drain, 0)


def kernel(inputs, pos_encoding):
    inputs = inputs.astype(jnp.int32)
    tab1 = pos_encoding[1:L + 1].reshape(WPR, 128)
    mesh = plsc.VectorSubcoreMesh(core_axis_name="c", subcore_axis_name="s")
    k = pl.kernel(
        _body,
        out_type=jax.ShapeDtypeStruct((B * WPR, 128), jnp.float32),
        mesh=mesh,
        compiler_params=pltpu.CompilerParams(use_tc_tiling_on_sc=False),
        scratch_types=[
            pltpu.VMEM((RPW, L), jnp.int32),
            pltpu.VMEM((2 * WPR, 128), jnp.float32),
            pltpu.VMEM((DIM,), jnp.float32),
            pltpu.VMEM((2 * WPR, 128), jnp.float32),
            pltpu.SemaphoreType.DMA,
            pltpu.SemaphoreType.DMA,
        ],
    )
    out = k(inputs, tab1, pos_encoding[0])
    return out.reshape(B, L, DIM)


# speedup vs baseline: 5.2921x; 5.2921x over previous
import jax
import jax.numpy as jnp
from jax import lax
from jax.experimental import pallas as pl
from jax.experimental.pallas import tpu as pltpu
from jax.experimental.pallas import tpu_sc as plsc

B, L = 4096, 200
DIM = 64
NC, NS = 2, 16
NW = NC * NS
LANES = 16
CH = 512
NR = B // CH
NJ = DIM // LANES
KMAX = (L + NW - 1) // NW


def _body(in_hbm, tab_hbm, r0_hbm, out_hbm,
          inrow_v, tab_v, r0_v, blk_v, dirty_v, sem0, sem1, sem_d):
    wid = lax.axis_index("s") * NC + lax.axis_index("c")

    pltpu.sync_copy(tab_hbm, tab_v)
    pltpu.sync_copy(r0_hbm, r0_v)

    r0c = [r0_v[pl.ds(j * LANES, LANES)] for j in range(NJ)]
    iota = jax.lax.iota(jnp.int32, LANES)
    idxd = [iota + j * LANES for j in range(NJ)]

    def drain0(n):
        def dr(i, c):
            pltpu.make_async_copy(blk_v.at[0],
                                  out_hbm.at[0, :, pl.ds(0, CH)], sem0).wait()
            return c
        lax.fori_loop(0, n, dr, 0)

    def drain1(n):
        def dr(i, c):
            pltpu.make_async_copy(blk_v.at[1],
                                  out_hbm.at[0, :, pl.ds(0, CH)], sem1).wait()
            return c
        lax.fori_loop(0, n, dr, 0)

    def do_l(k, cs):
        c0, c1 = cs
        l = wid + NW * k
        par = k % 2

        def run(cs):
            c0, c1 = cs

            def dpar0(c):
                drain0(c)
                return jnp.int32(0)

            def dpar1(c):
                drain1(c)
                return jnp.int32(0)
            lax.cond(par == 0, dpar0, dpar1, jnp.where(par == 0, c0, c1))

            pltpu.sync_copy(in_hbm.at[l], inrow_v)
            l2 = l // 2
            cb = pl.multiple_of((l % 2) * DIM, LANES)

            for j in range(NJ):
                tc = tab_v[l2, pl.ds(cb + j * LANES, LANES)]
                for lane in range(LANES):
                    d = j * LANES + lane
                    bs = jnp.broadcast_to(tc[lane], (LANES,))

                    def bw(c2, cc):
                        off = pl.multiple_of(c2 * 64, 64)
                        for q in range(4):
                            blk_v[par, d, pl.ds(off + q * LANES, LANES)] = bs
                        return cc
                    lax.fori_loop(0, CH // 64, bw, 0)

            def rng(r, cc):
                roff = pl.multiple_of(r * CH, CH)
                acc = jnp.zeros((LANES,), jnp.int32)
                for i in range(CH // LANES):
                    row32 = r * (CH // 128) + i // 8
                    v = inrow_v[row32, pl.ds((i % 8) * LANES, LANES)]
                    acc = acc | jnp.where(v == 0, 1, 0)
                s = acc[0]
                for i in range(1, LANES):
                    s = s | acc[i]

                def clean(c3):
                    def st0(c4):
                        pltpu.make_async_copy(
                            blk_v.at[0], out_hbm.at[l, :, pl.ds(roff, CH)],
                            sem0).start()
                        return c4

                    def st1(c4):
                        pltpu.make_async_copy(
                            blk_v.at[1], out_hbm.at[l, :, pl.ds(roff, CH)],
                            sem1).start()
                        return c4
                    return lax.cond(par == 0, st0, st1, c3) + 1

                def dirty(c3):
                    def cpy(c2, cc2):
                        d = c2 // (CH // LANES)
                        mino = pl.multiple_of((c2 % (CH // LANES)) * LANES,
                                              LANES)
                        dirty_v[d, pl.ds(mino, LANES)] = (
                            blk_v[par, d, pl.ds(mino, LANES)])
                        return cc2
                    lax.fori_loop(0, DIM * CH // LANES, cpy, 0)

                    def fixc(ch, cc2):
                        row32 = r * (CH // 128) + ch // 8
                        mino = pl.multiple_of((ch % 8) * LANES, LANES)
                        v = inrow_v[row32, pl.ds(mino, LANES)]
                        acc2 = jnp.where(v == 0, 1, 0)
                        s2 = acc2[0]
                        for i in range(1, LANES):
                            s2 = s2 | acc2[i]

                        @pl.when(s2 != 0)
                        def _():
                            col = pl.multiple_of(ch * LANES, LANES)
                            for j in range(NJ):
                                tc = tab_v[l2, pl.ds(cb + j * LANES, LANES)]
                                for lane in range(LANES):
                                    d = j * LANES + lane
                                    bt = jnp.broadcast_to(tc[lane], (LANES,))
                                    br = jnp.broadcast_to(r0c[j][lane],
                                                          (LANES,))
                                    dirty_v[d, pl.ds(col, LANES)] = (
                                        jnp.where(v == 0, br, bt))
                        return cc2
                    lax.fori_loop(0, CH // LANES, fixc, 0)

                    cp = pltpu.make_async_copy(
                        dirty_v, out_hbm.at[l, :, pl.ds(roff, CH)], sem_d)
                    cp.start()
                    cp.wait()
                    return c3

                return lax.cond(s == 0, clean, dirty, cc)

            cnt = lax.fori_loop(0, NR, rng, jnp.int32(0))
            c0n = jnp.where(par == 0, cnt, c0)
            c1n = jnp.where(par == 0, c1, cnt)
            return (c0n, c1n)

        return lax.cond(l < L, run, lambda cs_: cs_, (c0, c1))

    c0, c1 = lax.fori_loop(0, KMAX, do_l, (jnp.int32(0), jnp.int32(0)))
    drain0(c0)
    drain1(c1)


def kernel(inputs, pos_encoding):
    inputs = inputs.astype(jnp.int32)
    in3 = inputs.T.reshape(L, B // 128, 128)
    tab = pos_encoding[1:L + 1].reshape(L // 2, 128)
    mesh = plsc.VectorSubcoreMesh(core_axis_name="c", subcore_axis_name="s")
    k = pl.kernel(
        _body,
        out_type=jax.ShapeDtypeStruct((L, DIM, B), jnp.float32),
        mesh=mesh,
        scratch_types=[
            pltpu.VMEM((B // 128, 128), jnp.int32),
            pltpu.VMEM((L // 2, 128), jnp.float32),
            pltpu.VMEM((DIM,), jnp.float32),
            pltpu.VMEM((2, DIM, CH), jnp.float32),
            pltpu.VMEM((DIM, CH), jnp.float32),
            pltpu.SemaphoreType.DMA,
            pltpu.SemaphoreType.DMA,
            pltpu.SemaphoreType.DMA,
        ],
    )
    out_t = k(in3, tab, pos_encoding[0])
    return jnp.transpose(out_t, (2, 0, 1))


# speedup vs baseline: 5.3060x; 1.0026x over previous
import jax
import jax.numpy as jnp
from jax import lax
from jax.experimental import pallas as pl
from jax.experimental.pallas import tpu as pltpu
from jax.experimental.pallas import tpu_sc as plsc

B, L = 4096, 200
DIM = 64
NC, NS = 2, 16
NW = NC * NS
LANES = 16
CH = 512
NR = B // CH
NJ = DIM // LANES
KMAX = (L + NW - 1) // NW


def _body(in_hbm, tab_hbm, r0_hbm, out_hbm,
          inrow_v, tab_v, r0_v, blk_v, dirty_v, sem0, sem1, sem_d):
    wid = lax.axis_index("s") * NC + lax.axis_index("c")

    pltpu.sync_copy(tab_hbm, tab_v)
    pltpu.sync_copy(r0_hbm, r0_v)

    r0c = [r0_v[pl.ds(j * LANES, LANES)] for j in range(NJ)]

    def drain0(n):
        def dr(i, c):
            pltpu.make_async_copy(blk_v.at[0],
                                  out_hbm.at[0, :, pl.ds(0, CH)], sem0).wait()
            return c
        lax.fori_loop(0, n, dr, 0)

    def drain1(n):
        def dr(i, c):
            pltpu.make_async_copy(blk_v.at[1],
                                  out_hbm.at[0, :, pl.ds(0, CH)], sem1).wait()
            return c
        lax.fori_loop(0, n, dr, 0)

    def do_l(k, cs):
        c0, c1 = cs
        l = wid + NW * k
        par = k % 2

        def run(cs):
            c0, c1 = cs

            def dpar0(c):
                drain0(c)
                return jnp.int32(0)

            def dpar1(c):
                drain1(c)
                return jnp.int32(0)
            lax.cond(par == 0, dpar0, dpar1, jnp.where(par == 0, c0, c1))

            pltpu.sync_copy(in_hbm.at[l], inrow_v)
            l2 = l // 2
            cb = pl.multiple_of((l % 2) * DIM, LANES)

            for j in range(NJ):
                tc = tab_v[l2, pl.ds(cb + j * LANES, LANES)]
                for lane in range(LANES):
                    d = j * LANES + lane
                    bs = jnp.broadcast_to(tc[lane], (LANES,))

                    def bw(c2, cc):
                        off = pl.multiple_of(c2 * 64, 64)
                        for q in range(4):
                            blk_v[par, d, pl.ds(off + q * LANES, LANES)] = bs
                        return cc
                    lax.fori_loop(0, CH // 64, bw, 0)

            def rng(r, cc):
                roff = pl.multiple_of(r * CH, CH)
                acc = jnp.zeros((LANES,), jnp.int32)
                for i in range(CH // LANES):
                    row32 = r * (CH // 128) + i // 8
                    v = inrow_v[row32, pl.ds((i % 8) * LANES, LANES)]
                    acc = acc | jnp.where(v == 0, 1, 0)
                s = acc[0]
                for i in range(1, LANES):
                    s = s | acc[i]

                def clean(c3):
                    def st0(c4):
                        pltpu.make_async_copy(
                            blk_v.at[0], out_hbm.at[l, :, pl.ds(roff, CH)],
                            sem0).start()
                        return c4

                    def st1(c4):
                        pltpu.make_async_copy(
                            blk_v.at[1], out_hbm.at[l, :, pl.ds(roff, CH)],
                            sem1).start()
                        return c4
                    return lax.cond(par == 0, st0, st1, c3) + 1

                def dirty(c3):
                    def cpy(c2, cc2):
                        d = c2 // (CH // LANES)
                        mino = pl.multiple_of((c2 % (CH // LANES)) * LANES,
                                              LANES)
                        dirty_v[d, pl.ds(mino, LANES)] = (
                            blk_v[par, d, pl.ds(mino, LANES)])
                        return cc2
                    lax.fori_loop(0, DIM * CH // LANES, cpy, 0)

                    def fixc(ch, cc2):
                        row32 = r * (CH // 128) + ch // 8
                        mino = pl.multiple_of((ch % 8) * LANES, LANES)
                        v = inrow_v[row32, pl.ds(mino, LANES)]
                        acc2 = jnp.where(v == 0, 1, 0)
                        s2 = acc2[0]
                        for i in range(1, LANES):
                            s2 = s2 | acc2[i]

                        @pl.when(s2 != 0)
                        def _():
                            col = pl.multiple_of(ch * LANES, LANES)
                            for j in range(NJ):
                                tc = tab_v[l2, pl.ds(cb + j * LANES, LANES)]
                                for lane in range(LANES):
                                    d = j * LANES + lane
                                    bt = jnp.broadcast_to(tc[lane], (LANES,))
                                    br = jnp.broadcast_to(r0c[j][lane],
                                                          (LANES,))
                                    dirty_v[d, pl.ds(col, LANES)] = (
                                        jnp.where(v == 0, br, bt))
                        return cc2
                    lax.fori_loop(0, CH // LANES, fixc, 0)

                    cp = pltpu.make_async_copy(
                        dirty_v, out_hbm.at[l, :, pl.ds(roff, CH)], sem_d)
                    cp.start()
                    cp.wait()
                    return c3

                return lax.cond(s == 0, clean, dirty, cc)

            cnt = lax.fori_loop(0, NR, rng, jnp.int32(0))
            c0n = jnp.where(par == 0, cnt, c0)
            c1n = jnp.where(par == 0, c1, cnt)
            return (c0n, c1n)

        return lax.cond(l < L, run, lambda cs_: cs_, (c0, c1))

    c0, c1 = lax.fori_loop(0, KMAX, do_l, (jnp.int32(0), jnp.int32(0)))
    drain0(c0)
    drain1(c1)


def kernel(inputs, pos_encoding):
    inputs = inputs.astype(jnp.int32)
    in3 = inputs.T.reshape(L, B // 128, 128)
    tab = pos_encoding[1:L + 1].reshape(L // 2, 128)
    mesh = plsc.VectorSubcoreMesh(core_axis_name="c", subcore_axis_name="s")
    k = pl.kernel(
        _body,
        out_type=jax.ShapeDtypeStruct((L, DIM, B), jnp.float32),
        mesh=mesh,
        scratch_types=[
            pltpu.VMEM((B // 128, 128), jnp.int32),
            pltpu.VMEM((L // 2, 128), jnp.float32),
            pltpu.VMEM((DIM,), jnp.float32),
            pltpu.VMEM((2, DIM, CH), jnp.float32),
            pltpu.VMEM((DIM, CH), jnp.float32),
            pltpu.SemaphoreType.DMA,
            pltpu.SemaphoreType.DMA,
            pltpu.SemaphoreType.DMA,
        ],
    )
    out_t = k(in3, tab, pos_encoding[0])
    return jnp.transpose(out_t, (2, 0, 1))
